# baseline (speedup 1.0000x reference)
import functools

import jax
import jax.numpy as jnp
from jax import lax
from jax.experimental import pallas as pl
from jax.experimental.pallas import tpu as pltpu
from jax.experimental.pallas import tpu_sc as plsc


def _sc_gather(emb, idx_flat):
    info = plsc.get_sparse_core_info()
    nc, ns = info.num_cores, info.num_subcores
    nw = nc * ns
    b = idx_flat.shape[0]
    d = emb.shape[1]
    assert b % (8 * nw) == 0
    b_per_w = b // nw
    mesh = plsc.VectorSubcoreMesh(core_axis_name="c", subcore_axis_name="s")

    @functools.partial(
        pl.kernel,
        mesh=mesh,
        out_type=jax.ShapeDtypeStruct((b, d), jnp.float32),
        scratch_types=[
            pltpu.VMEM((b_per_w,), jnp.int32),
            pltpu.VMEM((b_per_w, d), jnp.float32),
            pltpu.SemaphoreType.DMA,
        ],
        compiler_params=pltpu.CompilerParams(use_tc_tiling_on_sc=False),
    )
    def gather_kernel(table_hbm, idx_hbm, out_hbm, idx_v, rows_v, sem):
        wid = lax.axis_index("s") * nc + lax.axis_index("c")
        base = wid * b_per_w
        pltpu.sync_copy(idx_hbm.at[pl.ds(base, b_per_w)], idx_v)
        pltpu.async_copy(table_hbm.at[idx_v], rows_v, sem).wait()
        pltpu.sync_copy(rows_v, out_hbm.at[pl.ds(base, b_per_w)])

    return gather_kernel(emb, idx_flat)


def _hidden_body(emb_ref, w1_ref, b1_ref, hidt_ref):
    h = lax.dot_general(
        w1_ref[...], emb_ref[...],
        (((1,), (1,)), ((), ())),
        preferred_element_type=jnp.float32,
    )
    hidt_ref[...] = jnp.maximum(h + b1_ref[...], 0.0)


def _hidden_t(embeds, w1, b1):
    batch = embeds.shape[0]
    hidden = w1.shape[0]
    return pl.pallas_call(
        _hidden_body,
        out_shape=jax.ShapeDtypeStruct((hidden, batch), jnp.float32),
    )(embeds, w1, b1.reshape(hidden, 1))


def _vocab_body(hidt_ref, w2_ref, b2_ref, outt_ref):
    out = lax.dot_general(
        w2_ref[...], hidt_ref[...],
        (((1,), (0,)), ((), ())),
        preferred_element_type=jnp.float32,
    )
    outt_ref[...] = out + b2_ref[...]


def _vocab_matmul_t(hidt, w2, b2, v_tile=2048):
    hidden, batch = hidt.shape
    vocab = w2.shape[0]
    nv = pl.cdiv(vocab, v_tile)
    return pl.pallas_call(
        _vocab_body,
        grid=(nv,),
        in_specs=[
            pl.BlockSpec(hidt.shape, lambda j: (0, 0)),
            pl.BlockSpec((v_tile, hidden), lambda j: (j, 0)),
            pl.BlockSpec((v_tile, 1), lambda j: (j, 0)),
        ],
        out_specs=pl.BlockSpec((v_tile, batch), lambda j: (j, 0)),
        out_shape=jax.ShapeDtypeStruct((vocab, batch), jnp.float32),
        compiler_params=pltpu.CompilerParams(
            dimension_semantics=("parallel",),
            vmem_limit_bytes=100 * 1024 * 1024,
        ),
    )(hidt, w2, b2.reshape(vocab, 1))


def kernel(inputs, emb, W1, b1, W2, b2):
    batch, context = inputs.shape
    d = emb.shape[1]
    idx_flat = inputs.reshape(-1)
    embeds = _sc_gather(emb, idx_flat)
    embeds = embeds.reshape(batch, context * d)
    hidt = _hidden_t(embeds, W1, b1)
    outt = _vocab_matmul_t(hidt, W2, b2)
    return outt.T


# speedup vs baseline: 1.0231x; 1.0231x over previous
import functools

import jax
import jax.numpy as jnp
from jax import lax
from jax.experimental import pallas as pl
from jax.experimental.pallas import tpu as pltpu
from jax.experimental.pallas import tpu_sc as plsc


def _sc_gather(emb, idx_flat):
    info = plsc.get_sparse_core_info()
    nc, ns = info.num_cores, info.num_subcores
    nw = nc * ns
    b = idx_flat.shape[0]
    d = emb.shape[1]
    assert b % (8 * nw) == 0
    b_per_w = b // nw
    mesh = plsc.VectorSubcoreMesh(core_axis_name="c", subcore_axis_name="s")

    @functools.partial(
        pl.kernel,
        mesh=mesh,
        out_type=jax.ShapeDtypeStruct((b, d), jnp.float32),
        scratch_types=[
            pltpu.VMEM((b_per_w,), jnp.int32),
            pltpu.VMEM((b_per_w, d), jnp.float32),
            pltpu.SemaphoreType.DMA,
        ],
        compiler_params=pltpu.CompilerParams(use_tc_tiling_on_sc=False),
    )
    def gather_kernel(table_hbm, idx_hbm, out_hbm, idx_v, rows_v, sem):
        wid = lax.axis_index("s") * nc + lax.axis_index("c")
        base = wid * b_per_w
        pltpu.sync_copy(idx_hbm.at[pl.ds(base, b_per_w)], idx_v)
        pltpu.async_copy(table_hbm.at[idx_v], rows_v, sem).wait()
        pltpu.sync_copy(rows_v, out_hbm.at[pl.ds(base, b_per_w)])

    return gather_kernel(emb, idx_flat)


def _mlp_body(emb_ref, w1_ref, b1_ref, w2_ref, b2_ref, outt_ref, hidt_ref):
    @pl.when(pl.program_id(0) == 0)
    def _():
        h = lax.dot_general(
            w1_ref[...], emb_ref[...],
            (((1,), (1,)), ((), ())),
            preferred_element_type=jnp.float32,
        )
        hidt_ref[...] = jnp.maximum(h + b1_ref[...], 0.0)

    out = lax.dot_general(
        w2_ref[...], hidt_ref[...],
        (((1,), (0,)), ((), ())),
        preferred_element_type=jnp.float32,
    )
    outt_ref[...] = out + b2_ref[...]


def _mlp_t(embeds, w1, b1, w2, b2, v_tile=4096):
    batch, cd = embeds.shape
    hidden = w1.shape[0]
    vocab = w2.shape[0]
    nv = pl.cdiv(vocab, v_tile)
    return pl.pallas_call(
        _mlp_body,
        grid=(nv,),
        in_specs=[
            pl.BlockSpec(embeds.shape, lambda j: (0, 0)),
            pl.BlockSpec(w1.shape, lambda j: (0, 0)),
            pl.BlockSpec((hidden, 1), lambda j: (0, 0)),
            pl.BlockSpec((v_tile, hidden), lambda j: (j, 0)),
            pl.BlockSpec((v_tile, 1), lambda j: (j, 0)),
        ],
        out_specs=pl.BlockSpec((v_tile, batch), lambda j: (j, 0)),
        out_shape=jax.ShapeDtypeStruct((vocab, batch), jnp.float32),
        scratch_shapes=[pltpu.VMEM((hidden, batch), jnp.float32)],
        compiler_params=pltpu.CompilerParams(
            dimension_semantics=("arbitrary",),
            vmem_limit_bytes=100 * 1024 * 1024,
        ),
    )(embeds, w1, b1.reshape(hidden, 1), w2, b2.reshape(vocab, 1))


def kernel(inputs, emb, W1, b1, W2, b2):
    batch, context = inputs.shape
    d = emb.shape[1]
    idx_flat = inputs.reshape(-1)
    embeds = _sc_gather(emb, idx_flat)
    embeds = embeds.reshape(batch, context * d)
    outt = _mlp_t(embeds, W1, b1, W2, b2)
    return outt.T
